# Initial kernel scaffold; baseline (speedup 1.0000x reference)
import functools

import jax
import jax.numpy as jnp
from jax import lax
from jax.experimental import pallas as pl
from jax.experimental.pallas import tpu as pltpu
from jax.experimental.pallas import tpu_sc as plsc

B = 16384
L = 200
EMBED = 16

NUM_CORES = 2
NUM_SUBCORES = 16
NW = NUM_CORES * NUM_SUBCORES
ROWS_PER_W = B // NW
CHUNK_ROWS = 8
CHUNK_IDX = CHUNK_ROWS * L
NCHUNKS = ROWS_PER_W // CHUNK_ROWS

_WINDOWS = []
_off = 0
while _off < CHUNK_IDX:
    _sz = min(128, CHUNK_IDX - _off)
    _WINDOWS.append((_off, _sz))
    _off += _sz


def _sc_pool(idx_flat, table):
    mesh = plsc.VectorSubcoreMesh(core_axis_name="c", subcore_axis_name="s")

    @functools.partial(
        pl.kernel,
        out_type=jax.ShapeDtypeStruct((B * EMBED,), jnp.float32),
        mesh=mesh,
        scratch_types=[
            pltpu.VMEM((CHUNK_IDX,), jnp.int32),
            pltpu.VMEM((CHUNK_IDX, EMBED), jnp.float32),
            pltpu.VMEM((ROWS_PER_W * EMBED,), jnp.float32),
            pltpu.SemaphoreType.DMA,
            pltpu.SemaphoreType.DMA,
        ],
    )
    def pool_kernel(idx_hbm, table_hbm, out_hbm, idx_v, rows_v, out_v, isem, gsem):
        wid = lax.axis_index("s") * NUM_CORES + lax.axis_index("c")
        ibase = wid * (ROWS_PER_W * L)
        obase = wid * (ROWS_PER_W * EMBED)
        inv_l = jnp.float32(1.0 / L)

        @pl.loop(0, NCHUNKS)
        def _(c):
            pltpu.async_copy(
                idx_hbm.at[pl.ds(ibase + c * CHUNK_IDX, CHUNK_IDX)], idx_v, isem
            ).wait()
            copies = [
                pltpu.async_copy(
                    table_hbm.at[idx_v.at[pl.ds(off, sz)]],
                    rows_v.at[pl.ds(off, sz)],
                    gsem,
                )
                for off, sz in _WINDOWS
            ]
            for cp in copies:
                cp.wait()
            for r in range(CHUNK_ROWS):
                rbase = r * L

                def body(k, accs, rbase=rbase):
                    a0, a1 = accs
                    j = rbase + k * 8
                    a0 += rows_v[j]
                    a1 += rows_v[j + 1]
                    a0 += rows_v[j + 2]
                    a1 += rows_v[j + 3]
                    a0 += rows_v[j + 4]
                    a1 += rows_v[j + 5]
                    a0 += rows_v[j + 6]
                    a1 += rows_v[j + 7]
                    return a0, a1

                z = jnp.zeros((EMBED,), jnp.float32)
                a0, a1 = lax.fori_loop(0, L // 8, body, (z, z))
                out_v[pl.ds((c * CHUNK_ROWS + r) * EMBED, EMBED)] = (a0 + a1) * inv_l

        pltpu.sync_copy(out_v, out_hbm.at[pl.ds(obase, ROWS_PER_W * EMBED)])

    return pool_kernel(idx_flat, table)


def _mlp_body(x_ref, w1_ref, b1_ref, w2_ref, b2_ref, o_ref):
    x = x_ref[...]
    h = jnp.dot(x, w1_ref[...], preferred_element_type=jnp.float32) + b1_ref[...]
    h = jnp.maximum(h, 0.0)
    z = jnp.dot(h, w2_ref[...], preferred_element_type=jnp.float32) + b2_ref[...]
    o_ref[...] = jax.nn.sigmoid(z)


def _tc_mlp(pooled, W1, b1, W2, b2):
    return pl.pallas_call(
        _mlp_body,
        out_shape=jax.ShapeDtypeStruct((B, 1), jnp.float32),
    )(pooled, W1, b1.reshape(1, EMBED), W2, b2.reshape(1, 1))


def kernel(inputs, table, W1, b1, W2, b2):
    idx_flat = inputs.reshape(-1).astype(jnp.int32)
    pooled = _sc_pool(idx_flat, table).reshape(B, EMBED)
    return _tc_mlp(pooled, W1, b1, W2, b2)


# speedup vs baseline: 40.1976x; 40.1976x over previous
import functools

import jax
import jax.numpy as jnp
from jax import lax
from jax.experimental import pallas as pl
from jax.experimental.pallas import tpu as pltpu
from jax.experimental.pallas import tpu_sc as plsc

B = 16384
L = 200
EMBED = 16

NUM_CORES = 2
NUM_SUBCORES = 16
NW = NUM_CORES * NUM_SUBCORES
ROWS_PER_W = B // NW
CHUNK_ROWS = 8
CHUNK_IDX = CHUNK_ROWS * L
NCHUNKS = ROWS_PER_W // CHUNK_ROWS

_WINDOWS = []
_off = 0
while _off < CHUNK_IDX:
    _sz = min(128, CHUNK_IDX - _off)
    _WINDOWS.append((_off, _sz))
    _off += _sz


def _sc_pool(idx_flat, table):
    mesh = plsc.VectorSubcoreMesh(core_axis_name="c", subcore_axis_name="s")

    @functools.partial(
        pl.kernel,
        out_type=jax.ShapeDtypeStruct((B * EMBED,), jnp.float32),
        mesh=mesh,
        scratch_types=[
            pltpu.VMEM((CHUNK_IDX,), jnp.int32),
            pltpu.VMEM((CHUNK_IDX, EMBED), jnp.float32),
            pltpu.VMEM((ROWS_PER_W * EMBED,), jnp.float32),
            pltpu.SemaphoreType.DMA,
            pltpu.SemaphoreType.DMA,
        ],
        compiler_params=pltpu.CompilerParams(use_tc_tiling_on_sc=False),
    )
    def pool_kernel(idx_hbm, table_hbm, out_hbm, idx_v, rows_v, out_v, isem, gsem):
        wid = lax.axis_index("s") * NUM_CORES + lax.axis_index("c")
        ibase = wid * (ROWS_PER_W * L)
        obase = wid * (ROWS_PER_W * EMBED)
        inv_l = jnp.float32(1.0 / L)

        @pl.loop(0, NCHUNKS)
        def _(c):
            pltpu.async_copy(
                idx_hbm.at[pl.ds(ibase + c * CHUNK_IDX, CHUNK_IDX)], idx_v, isem
            ).wait()
            copies = [
                pltpu.async_copy(
                    table_hbm.at[idx_v.at[pl.ds(off, sz)]],
                    rows_v.at[pl.ds(off, sz)],
                    gsem,
                )
                for off, sz in _WINDOWS
            ]
            for cp in copies:
                cp.wait()
            for r in range(CHUNK_ROWS):
                rbase = r * L

                def body(k, accs, rbase=rbase):
                    a0, a1 = accs
                    j = rbase + k * 8
                    a0 += rows_v[j]
                    a1 += rows_v[j + 1]
                    a0 += rows_v[j + 2]
                    a1 += rows_v[j + 3]
                    a0 += rows_v[j + 4]
                    a1 += rows_v[j + 5]
                    a0 += rows_v[j + 6]
                    a1 += rows_v[j + 7]
                    return a0, a1

                z = jnp.zeros((EMBED,), jnp.float32)
                a0, a1 = lax.fori_loop(0, L // 8, body, (z, z))
                out_v[pl.ds((c * CHUNK_ROWS + r) * EMBED, EMBED)] = (a0 + a1) * inv_l

        pltpu.sync_copy(out_v, out_hbm.at[pl.ds(obase, ROWS_PER_W * EMBED)])

    return pool_kernel(idx_flat, table)


def _mlp_body(x_ref, w1_ref, b1_ref, w2_ref, b2_ref, o_ref):
    x = x_ref[...]
    h = jnp.dot(x, w1_ref[...], preferred_element_type=jnp.float32) + b1_ref[...]
    h = jnp.maximum(h, 0.0)
    z = jnp.dot(h, w2_ref[...], preferred_element_type=jnp.float32) + b2_ref[...]
    o_ref[...] = jax.nn.sigmoid(z)


def _tc_mlp(pooled, W1, b1, W2, b2):
    return pl.pallas_call(
        _mlp_body,
        out_shape=jax.ShapeDtypeStruct((B, 1), jnp.float32),
    )(pooled, W1, b1.reshape(1, EMBED), W2, b2.reshape(1, 1))


def kernel(inputs, table, W1, b1, W2, b2):
    idx_flat = inputs.reshape(-1).astype(jnp.int32)
    pooled = _sc_pool(idx_flat, table).reshape(B, EMBED)
    return _tc_mlp(pooled, W1, b1, W2, b2)


# speedup vs baseline: 56.9537x; 1.4168x over previous
import functools

import jax
import jax.numpy as jnp
from jax import lax
from jax.experimental import pallas as pl
from jax.experimental.pallas import tpu as pltpu
from jax.experimental.pallas import tpu_sc as plsc

B = 16384
L = 200
EMBED = 16

NUM_CORES = 2
NUM_SUBCORES = 16
NW = NUM_CORES * NUM_SUBCORES
ROWS_PER_W = B // NW
CHUNK_ROWS = 8
CHUNK_IDX = CHUNK_ROWS * L
NCHUNKS = ROWS_PER_W // CHUNK_ROWS

_WINDOWS = []
_off = 0
while _off < CHUNK_IDX:
    _sz = min(128, CHUNK_IDX - _off)
    _WINDOWS.append((_off, _sz))
    _off += _sz


def _sc_pool(idx_flat, table):
    mesh = plsc.VectorSubcoreMesh(core_axis_name="c", subcore_axis_name="s")

    @functools.partial(
        pl.kernel,
        out_type=jax.ShapeDtypeStruct((B * EMBED,), jnp.float32),
        mesh=mesh,
        scratch_types=[
            pltpu.VMEM((CHUNK_IDX,), jnp.int32),
            pltpu.VMEM((CHUNK_IDX,), jnp.int32),
            pltpu.VMEM((CHUNK_IDX, EMBED), jnp.float32),
            pltpu.VMEM((CHUNK_IDX, EMBED), jnp.float32),
            pltpu.VMEM((ROWS_PER_W * EMBED,), jnp.float32),
            pltpu.SemaphoreType.DMA,
            pltpu.SemaphoreType.DMA,
            pltpu.SemaphoreType.DMA,
        ],
        compiler_params=pltpu.CompilerParams(use_tc_tiling_on_sc=False),
    )
    def pool_kernel(idx_hbm, table_hbm, out_hbm,
                    idx0, idx1, rows0, rows1, out_v, isem, gsem0, gsem1):
        wid = lax.axis_index("s") * NUM_CORES + lax.axis_index("c")
        ibase = wid * (ROWS_PER_W * L)
        obase = wid * (ROWS_PER_W * EMBED)
        inv_l = jnp.float32(1.0 / L)

        def fire_gathers(idx_v, rows_v, gsem):
            for off, sz in _WINDOWS:
                pltpu.async_copy(
                    table_hbm.at[idx_v.at[pl.ds(off, sz)]],
                    rows_v.at[pl.ds(off, sz)],
                    gsem,
                )

        def drain_gathers(rows_v, gsem):
            pltpu.make_async_copy(
                table_hbm.at[pl.ds(0, CHUNK_IDX)], rows_v, gsem
            ).wait()

        def start_idx(c, idx_v):
            pltpu.async_copy(
                idx_hbm.at[pl.ds(ibase + c * CHUNK_IDX, CHUNK_IDX)], idx_v, isem
            )

        def wait_idx(idx_v):
            pltpu.make_async_copy(
                idx_hbm.at[pl.ds(0, CHUNK_IDX)], idx_v, isem
            ).wait()

        def accumulate(rows_v, c):
            for r in range(CHUNK_ROWS):
                rbase = r * L

                def body(k, accs, rbase=rbase):
                    a0, a1, a2, a3 = accs
                    j = rbase + k * 25
                    for u in range(25):
                        v = rows_v[j + u]
                        if u % 4 == 0:
                            a0 += v
                        elif u % 4 == 1:
                            a1 += v
                        elif u % 4 == 2:
                            a2 += v
                        else:
                            a3 += v
                    return a0, a1, a2, a3

                z = jnp.zeros((EMBED,), jnp.float32)
                a0, a1, a2, a3 = lax.fori_loop(0, L // 25, body, (z, z, z, z))
                out_v[pl.ds((c * CHUNK_ROWS + r) * EMBED, EMBED)] = (
                    (a0 + a1) + (a2 + a3)
                ) * inv_l

        start_idx(0, idx0)
        wait_idx(idx0)
        fire_gathers(idx0, rows0, gsem0)
        start_idx(1, idx1)

        @pl.loop(0, NCHUNKS, step=2)
        def _(i):
            for p, (ic, inx, rc, rnx, gc, gnx) in enumerate(
                ((idx0, idx1, rows0, rows1, gsem0, gsem1),
                 (idx1, idx0, rows1, rows0, gsem1, gsem0))
            ):
                c = i + p

                @pl.when(c + 1 < NCHUNKS)
                def _():
                    wait_idx(inx)
                    fire_gathers(inx, rnx, gnx)

                drain_gathers(rc, gc)

                @pl.when(c + 2 < NCHUNKS)
                def _():
                    start_idx(c + 2, ic)

                accumulate(rc, c)

        pltpu.sync_copy(out_v, out_hbm.at[pl.ds(obase, ROWS_PER_W * EMBED)])

    return pool_kernel(idx_flat, table)


def _mlp_body(x_ref, w1_ref, b1_ref, w2_ref, b2_ref, o_ref):
    x = x_ref[...]
    h = jnp.dot(x, w1_ref[...], preferred_element_type=jnp.float32) + b1_ref[...]
    h = jnp.maximum(h, 0.0)
    z = jnp.dot(h, w2_ref[...], preferred_element_type=jnp.float32) + b2_ref[...]
    o_ref[...] = jax.nn.sigmoid(z)


def _tc_mlp(pooled, W1, b1, W2, b2):
    return pl.pallas_call(
        _mlp_body,
        out_shape=jax.ShapeDtypeStruct((B, 1), jnp.float32),
    )(pooled, W1, b1.reshape(1, EMBED), W2, b2.reshape(1, 1))


def kernel(inputs, table, W1, b1, W2, b2):
    idx_flat = inputs.reshape(-1).astype(jnp.int32)
    pooled = _sc_pool(idx_flat, table).reshape(B, EMBED)
    return _tc_mlp(pooled, W1, b1, W2, b2)


# speedup vs baseline: 57.0124x; 1.0010x over previous
import functools

import jax
import jax.numpy as jnp
from jax import lax
from jax.experimental import pallas as pl
from jax.experimental.pallas import tpu as pltpu
from jax.experimental.pallas import tpu_sc as plsc

B = 16384
L = 200
EMBED = 16

NUM_CORES = 2
NUM_SUBCORES = 16
NW = NUM_CORES * NUM_SUBCORES
ROWS_PER_W = B // NW
CHUNK_ROWS = 8
CHUNK_IDX = CHUNK_ROWS * L
NCHUNKS = ROWS_PER_W // CHUNK_ROWS

_WINDOW_SZ = CHUNK_IDX
_WINDOWS = []
_off = 0
while _off < CHUNK_IDX:
    _sz = min(_WINDOW_SZ, CHUNK_IDX - _off)
    _WINDOWS.append((_off, _sz))
    _off += _sz


def _sc_pool(idx_flat, table):
    mesh = plsc.VectorSubcoreMesh(core_axis_name="c", subcore_axis_name="s")

    @functools.partial(
        pl.kernel,
        out_type=jax.ShapeDtypeStruct((B * EMBED,), jnp.float32),
        mesh=mesh,
        scratch_types=[
            pltpu.VMEM((CHUNK_IDX,), jnp.int32),
            pltpu.VMEM((CHUNK_IDX,), jnp.int32),
            pltpu.VMEM((CHUNK_IDX, EMBED), jnp.float32),
            pltpu.VMEM((CHUNK_IDX, EMBED), jnp.float32),
            pltpu.VMEM((ROWS_PER_W * EMBED,), jnp.float32),
            pltpu.SemaphoreType.DMA,
            pltpu.SemaphoreType.DMA,
            pltpu.SemaphoreType.DMA,
        ],
        compiler_params=pltpu.CompilerParams(use_tc_tiling_on_sc=False),
    )
    def pool_kernel(idx_hbm, table_hbm, out_hbm,
                    idx0, idx1, rows0, rows1, out_v, isem, gsem0, gsem1):
        wid = lax.axis_index("s") * NUM_CORES + lax.axis_index("c")
        ibase = wid * (ROWS_PER_W * L)
        obase = wid * (ROWS_PER_W * EMBED)
        inv_l = jnp.float32(1.0 / L)

        def fire_gathers(idx_v, rows_v, gsem):
            for off, sz in _WINDOWS:
                pltpu.async_copy(
                    table_hbm.at[idx_v.at[pl.ds(off, sz)]],
                    rows_v.at[pl.ds(off, sz)],
                    gsem,
                )

        def drain_gathers(rows_v, gsem):
            pltpu.make_async_copy(
                table_hbm.at[pl.ds(0, CHUNK_IDX)], rows_v, gsem
            ).wait()

        def start_idx(c, idx_v):
            pltpu.async_copy(
                idx_hbm.at[pl.ds(ibase + c * CHUNK_IDX, CHUNK_IDX)], idx_v, isem
            )

        def wait_idx(idx_v):
            pltpu.make_async_copy(
                idx_hbm.at[pl.ds(0, CHUNK_IDX)], idx_v, isem
            ).wait()

        def accumulate(rows_v, c):
            for r in range(CHUNK_ROWS):
                rbase = r * L

                def body(k, accs, rbase=rbase):
                    a0, a1, a2, a3 = accs
                    j = rbase + k * 25
                    for u in range(25):
                        v = rows_v[j + u]
                        if u % 4 == 0:
                            a0 += v
                        elif u % 4 == 1:
                            a1 += v
                        elif u % 4 == 2:
                            a2 += v
                        else:
                            a3 += v
                    return a0, a1, a2, a3

                z = jnp.zeros((EMBED,), jnp.float32)
                a0, a1, a2, a3 = lax.fori_loop(0, L // 25, body, (z, z, z, z))
                out_v[pl.ds((c * CHUNK_ROWS + r) * EMBED, EMBED)] = (
                    (a0 + a1) + (a2 + a3)
                ) * inv_l

        start_idx(0, idx0)
        wait_idx(idx0)
        fire_gathers(idx0, rows0, gsem0)
        start_idx(1, idx1)

        @pl.loop(0, NCHUNKS, step=2)
        def _(i):
            for p, (ic, inx, rc, rnx, gc, gnx) in enumerate(
                ((idx0, idx1, rows0, rows1, gsem0, gsem1),
                 (idx1, idx0, rows1, rows0, gsem1, gsem0))
            ):
                c = i + p

                @pl.when(c + 1 < NCHUNKS)
                def _():
                    wait_idx(inx)
                    fire_gathers(inx, rnx, gnx)

                drain_gathers(rc, gc)

                @pl.when(c + 2 < NCHUNKS)
                def _():
                    start_idx(c + 2, ic)

                accumulate(rc, c)

        pltpu.sync_copy(out_v, out_hbm.at[pl.ds(obase, ROWS_PER_W * EMBED)])

    return pool_kernel(idx_flat, table)


def _mlp_body(x_ref, w1_ref, b1_ref, w2_ref, b2_ref, o_ref):
    x = x_ref[...]
    h = jnp.dot(x, w1_ref[...], preferred_element_type=jnp.float32) + b1_ref[...]
    h = jnp.maximum(h, 0.0)
    z = jnp.dot(h, w2_ref[...], preferred_element_type=jnp.float32) + b2_ref[...]
    o_ref[...] = jax.nn.sigmoid(z)


def _tc_mlp(pooled, W1, b1, W2, b2):
    return pl.pallas_call(
        _mlp_body,
        out_shape=jax.ShapeDtypeStruct((B, 1), jnp.float32),
    )(pooled, W1, b1.reshape(1, EMBED), W2, b2.reshape(1, 1))


def kernel(inputs, table, W1, b1, W2, b2):
    idx_flat = inputs.reshape(-1).astype(jnp.int32)
    pooled = _sc_pool(idx_flat, table).reshape(B, EMBED)
    return _tc_mlp(pooled, W1, b1, W2, b2)
